# Initial kernel scaffold; baseline (speedup 1.0000x reference)
import jax
import jax.numpy as jnp
from jax.experimental import pallas as pl


def kernel(ent, ent_type, batch_type, ent_table, type_table):
    raise NotImplementedError("write your pallas kernel here")


# speedup vs baseline: 4.5886x; 4.5886x over previous
import jax
import jax.numpy as jnp
from jax import lax
from jax.experimental import pallas as pl
from jax.experimental.pallas import tpu as pltpu
from jax.experimental.pallas import tpu_sc as plsc

NC, NS, L = 2, 16, 16
NW = NC * NS
B = 16384
NEG = 50
DIM = 64
NT = 1000
NTP = 1008
BPW = B // NW
C = 128
NCHUNK = BPW // C


_GATHER_DNUMS = lax.GatherDimensionNumbers(
    offset_dims=(), collapsed_slice_dims=(0,), start_index_map=(0,))


def _lane_gather(vec, idx):
    return lax.gather(vec, idx[:, None], _GATHER_DNUMS, (1,),
                      mode=lax.GatherScatterMode.PROMISE_IN_BOUNDS)


def _rsqrt(x):
    i = lax.bitcast_convert_type(x, jnp.int32)
    i = jnp.int32(0x5F3759DF) - lax.shift_right_logical(i, 1)
    y = lax.bitcast_convert_type(i, jnp.float32)
    for _ in range(3):
        y = y * (1.5 - 0.5 * x * y * y)
    return y


def _sc_kernel(tt_hbm, ent_hbm, et_hbm, table_hbm, out_hbm,
               tt_v, inv_t_v, idx_v, rows_v, et_v, inv_e_v, out_v, sem):
    wid = lax.axis_index("s") * NC + lax.axis_index("c")

    pltpu.sync_copy(tt_hbm, tt_v)

    def tnorm_body(cb, _):
        acc = jnp.zeros((L,), jnp.float32)
        for j in range(DIM):
            v = tt_v[j, pl.ds(cb * L, L)]
            acc = acc + v * v
        inv_t_v[pl.ds(cb * L, L)] = _rsqrt(jnp.maximum(acc, 1e-24))
        return 0
    lax.fori_loop(0, NTP // L, tnorm_body, 0)

    lane_iota = lax.iota(jnp.int32, L)
    lane_consts = [jnp.full((L,), i, jnp.int32) for i in range(L)]

    def chunk_body(cnk, _):
        base = wid * BPW + cnk * C
        pltpu.sync_copy(ent_hbm.at[pl.ds(base, C)], idx_v)
        cp = pltpu.async_copy(table_hbm.at[idx_v], rows_v, sem)
        pltpu.sync_copy(et_hbm.at[pl.ds(base * NEG, C * NEG)],
                        et_v.at[pl.ds(0, C * NEG)])
        et_v[pl.ds(C * NEG, L)] = jnp.zeros((L,), jnp.int32)
        cp.wait()

        def enorm_body(rc, _):
            rid = jnp.full((L,), rc * L, jnp.int32) + lane_iota
            acc = jnp.zeros((L,), jnp.float32)
            for j in range(DIM):
                v = plsc.load_gather(rows_v, [rid, jnp.full((L,), j, jnp.int32)])
                acc = acc + v * v
            inv_e_v[pl.ds(rc * L, L)] = _rsqrt(jnp.maximum(acc, 1e-24))
            return 0
        lax.fori_loop(0, C // L, enorm_body, 0)

        def b_body(b, _):
            bsplat = jnp.full((L,), b, jnp.int32)
            inv_e_b = plsc.load_gather(inv_e_v, [bsplat])
            for g in range(4):
                off = b * NEG + g * L
                tvec = et_v[pl.ds(off, L)]
                acc = jnp.zeros((L,), jnp.float32)
                for j in range(DIM):
                    e_b = plsc.load_gather(
                        rows_v, [bsplat, jnp.full((L,), j, jnp.int32)])
                    col = plsc.load_gather(
                        tt_v, [jnp.full((L,), j, jnp.int32), tvec])
                    acc = acc + e_b * col
                itv = plsc.load_gather(inv_t_v, [tvec])
                d = acc * itv * inv_e_b
                s = jnp.maximum(2.0 - 2.0 * d, 0.0)
                out_v[pl.ds(off, L)] = s * _rsqrt(jnp.maximum(s, 1e-30))
            return 0
        lax.fori_loop(0, C, b_body, 0)

        pltpu.sync_copy(out_v.at[pl.ds(0, C * NEG)],
                        out_hbm.at[pl.ds(base * NEG, C * NEG)])
        return 0
    lax.fori_loop(0, NCHUNK, chunk_body, 0)


def kernel(ent, ent_type, batch_type, ent_table, type_table):
    tt = jnp.pad(type_table.astype(jnp.float32).T, ((0, 0), (0, NTP - NT)))
    ent_i = ent.astype(jnp.int32)
    et_flat = ent_type.astype(jnp.int32).reshape(-1)

    mesh = plsc.VectorSubcoreMesh(core_axis_name="c", subcore_axis_name="s",
                                  num_cores=NC, num_subcores=NS)
    run = pl.kernel(
        _sc_kernel,
        out_type=jax.ShapeDtypeStruct((B * NEG,), jnp.float32),
        mesh=mesh,
        compiler_params=pltpu.CompilerParams(needs_layout_passes=False,
                                             use_tc_tiling_on_sc=False),
        scratch_types=[
            pltpu.VMEM((DIM, NTP), jnp.float32),
            pltpu.VMEM((NTP,), jnp.float32),
            pltpu.VMEM((C,), jnp.int32),
            pltpu.VMEM((C, DIM), jnp.float32),
            pltpu.VMEM((C * NEG + L,), jnp.int32),
            pltpu.VMEM((C,), jnp.float32),
            pltpu.VMEM((C * NEG + L,), jnp.float32),
            pltpu.SemaphoreType.DMA,
        ],
    )
    out = run(tt, ent_i, et_flat, ent_table.astype(jnp.float32))
    return out.reshape(B, NEG)


# speedup vs baseline: 5.4152x; 1.1801x over previous
import jax
import jax.numpy as jnp
from jax import lax
from jax.experimental import pallas as pl
from jax.experimental.pallas import tpu as pltpu
from jax.experimental.pallas import tpu_sc as plsc

NC, NS, L = 2, 16, 16
NW = NC * NS
B = 16384
NEG = 50
DIM = 64
NT = 1000
NTP = 1008
BPW = B // NW
C = 128
NCHUNK = BPW // C


_GATHER_DNUMS = lax.GatherDimensionNumbers(
    offset_dims=(), collapsed_slice_dims=(0,), start_index_map=(0,))


def _lane_gather(vec, idx):
    return lax.gather(vec, idx[:, None], _GATHER_DNUMS, (1,),
                      mode=lax.GatherScatterMode.PROMISE_IN_BOUNDS)


def _rsqrt(x):
    i = lax.bitcast_convert_type(x, jnp.int32)
    i = jnp.int32(0x5F3759DF) - lax.shift_right_logical(i, 1)
    y = lax.bitcast_convert_type(i, jnp.float32)
    for _ in range(3):
        y = y * (1.5 - 0.5 * x * y * y)
    return y


def _sc_kernel(tt_hbm, ent_hbm, et_hbm, table_hbm, out_hbm,
               tt_v, inv_t_v, idx_v, rows_v, et_v, inv_e_v, out_v, sem):
    wid = lax.axis_index("s") * NC + lax.axis_index("c")

    pltpu.sync_copy(tt_hbm, tt_v)

    def tnorm_body(cb, _):
        acc = jnp.zeros((L,), jnp.float32)
        for j in range(DIM):
            v = tt_v[j, pl.ds(cb * L, L)]
            acc = acc + v * v
        inv_t_v[pl.ds(cb * L, L)] = _rsqrt(jnp.maximum(acc, 1e-24))
        return 0
    lax.fori_loop(0, NTP // L, tnorm_body, 0)

    lane_iota = lax.iota(jnp.int32, L)
    lane_consts = [jnp.full((L,), i, jnp.int32) for i in range(L)]

    def chunk_body(cnk, _):
        base = wid * BPW + cnk * C
        pltpu.sync_copy(ent_hbm.at[pl.ds(base, C)], idx_v)
        cp = pltpu.async_copy(table_hbm.at[idx_v], rows_v, sem)
        pltpu.sync_copy(et_hbm.at[pl.ds(base * NEG, C * NEG)],
                        et_v.at[pl.ds(0, C * NEG)])
        et_v[pl.ds(C * NEG, L)] = jnp.zeros((L,), jnp.int32)
        cp.wait()

        def enorm_body(rc, _):
            rid = jnp.full((L,), rc * L, jnp.int32) + lane_iota
            acc = jnp.zeros((L,), jnp.float32)
            for j in range(DIM):
                v = plsc.load_gather(rows_v, [rid, jnp.full((L,), j, jnp.int32)])
                acc = acc + v * v
            inv_e_v[pl.ds(rc * L, L)] = _rsqrt(jnp.maximum(acc, 1e-24))
            return 0
        lax.fori_loop(0, C // L, enorm_body, 0)

        def b_body(b, _):
            ev = [rows_v[b, pl.ds(k * L, L)] for k in range(DIM // L)]
            bsplat = jnp.full((L,), b, jnp.int32)
            inv_e_b = plsc.load_gather(inv_e_v, [bsplat])
            off = b * NEG
            tvecs = [et_v[pl.ds(off + g * L, L)] for g in range(4)]
            accs = [jnp.zeros((L,), jnp.float32) for _ in range(4)]
            for j in range(DIM):
                e_b = _lane_gather(ev[j // L], lane_consts[j % L])
                jsplat = jnp.full((L,), j, jnp.int32)
                for g in range(4):
                    accs[g] = accs[g] + e_b * plsc.load_gather(
                        tt_v, [jsplat, tvecs[g]])
            for g in range(4):
                itv = plsc.load_gather(inv_t_v, [tvecs[g]])
                d = accs[g] * itv * inv_e_b
                s = jnp.maximum(2.0 - 2.0 * d, 0.0)
                out_v[pl.ds(off + g * L, L)] = s * _rsqrt(jnp.maximum(s, 1e-30))
            return 0
        lax.fori_loop(0, C, b_body, 0)

        pltpu.sync_copy(out_v.at[pl.ds(0, C * NEG)],
                        out_hbm.at[pl.ds(base * NEG, C * NEG)])
        return 0
    lax.fori_loop(0, NCHUNK, chunk_body, 0)


def kernel(ent, ent_type, batch_type, ent_table, type_table):
    tt = jnp.pad(type_table.astype(jnp.float32).T, ((0, 0), (0, NTP - NT)))
    ent_i = ent.astype(jnp.int32)
    et_flat = ent_type.astype(jnp.int32).reshape(-1)

    mesh = plsc.VectorSubcoreMesh(core_axis_name="c", subcore_axis_name="s",
                                  num_cores=NC, num_subcores=NS)
    run = pl.kernel(
        _sc_kernel,
        out_type=jax.ShapeDtypeStruct((B * NEG,), jnp.float32),
        mesh=mesh,
        compiler_params=pltpu.CompilerParams(needs_layout_passes=False,
                                             use_tc_tiling_on_sc=False),
        scratch_types=[
            pltpu.VMEM((DIM, NTP), jnp.float32),
            pltpu.VMEM((NTP,), jnp.float32),
            pltpu.VMEM((C,), jnp.int32),
            pltpu.VMEM((C, DIM), jnp.float32),
            pltpu.VMEM((C * NEG + L,), jnp.int32),
            pltpu.VMEM((C,), jnp.float32),
            pltpu.VMEM((C * NEG + L,), jnp.float32),
            pltpu.SemaphoreType.DMA,
        ],
    )
    out = run(tt, ent_i, et_flat, ent_table.astype(jnp.float32))
    return out.reshape(B, NEG)


# speedup vs baseline: 5.6471x; 1.0428x over previous
import jax
import jax.numpy as jnp
from jax import lax
from jax.experimental import pallas as pl
from jax.experimental.pallas import tpu as pltpu
from jax.experimental.pallas import tpu_sc as plsc

NC, NS, L = 2, 16, 16
NW = NC * NS
B = 16384
NEG = 50
DIM = 64
NT = 1000
NTP = 1024
NPAIR = DIM // 2
BPW = B // NW
C = 128
NCHUNK = BPW // C

def _rsqrt(x):
    i = lax.bitcast_convert_type(x, jnp.int32)
    i = jnp.int32(0x5F3759DF) - lax.shift_right_logical(i, 1)
    y = lax.bitcast_convert_type(i, jnp.float32)
    for _ in range(3):
        y = y * (1.5 - 0.5 * x * y * y)
    return y


_GATHER_DNUMS = lax.GatherDimensionNumbers(
    offset_dims=(), collapsed_slice_dims=(0,), start_index_map=(0,))


def _lane_gather(vec, idx):
    return lax.gather(vec, idx[:, None], _GATHER_DNUMS, (1,),
                      mode=lax.GatherScatterMode.PROMISE_IN_BOUNDS)


def _unpack_lo(w):
    return lax.bitcast_convert_type(lax.shift_left(w, 16), jnp.float32)


def _unpack_hi(w):
    return lax.bitcast_convert_type(
        jnp.bitwise_and(w, jnp.int32(-65536)), jnp.float32)


def _sc_kernel(ttb_hbm, ent_hbm, et_hbm, table_hbm, out_hbm,
               ttb_v, inv_t_v, idx_v, rows_v, et_v, out_v, sem):
    wid = lax.axis_index("s") * NC + lax.axis_index("c")

    pltpu.sync_copy(ttb_hbm, ttb_v)

    def tnorm_body(cb, _):
        acc = jnp.zeros((L,), jnp.float32)
        for j2 in range(NPAIR):
            w = ttb_v[j2, pl.ds(cb * L, L)]
            lo = _unpack_lo(w)
            hi = _unpack_hi(w)
            acc = acc + lo * lo + hi * hi
        inv_t_v[pl.ds(cb * L, L)] = _rsqrt(jnp.maximum(acc, 1e-24))
        return 0
    lax.fori_loop(0, NTP // L, tnorm_body, 0)

    lane_consts = [jnp.full((L,), i, jnp.int32) for i in range(L)]

    def chunk_body(cnk, _):
        base = wid * BPW + cnk * C
        pltpu.sync_copy(ent_hbm.at[pl.ds(base, C)], idx_v)
        cp = pltpu.async_copy(table_hbm.at[idx_v], rows_v, sem)
        pltpu.sync_copy(et_hbm.at[pl.ds(base * NEG, C * NEG)],
                        et_v.at[pl.ds(0, C * NEG)])
        et_v[pl.ds(C * NEG, L)] = jnp.zeros((L,), jnp.int32)
        cp.wait()

        def b_body(b, _):
            ev = [rows_v[b, pl.ds(k * L, L)] for k in range(DIM // L)]
            sq = ev[0] * ev[0] + ev[1] * ev[1] + ev[2] * ev[2] + ev[3] * ev[3]
            tot = _lane_gather(plsc.cumsum(sq), lane_consts[L - 1])
            inv_e_b = _rsqrt(jnp.maximum(tot, 1e-24))

            off = b * NEG
            tvecs = [et_v[pl.ds(off + g * L, L)] for g in range(4)]
            acc_e = [jnp.zeros((L,), jnp.float32) for _ in range(4)]
            acc_o = [jnp.zeros((L,), jnp.float32) for _ in range(4)]
            for j2 in range(NPAIR):
                e_even = _lane_gather(ev[j2 // 8], lane_consts[(2 * j2) % L])
                e_odd = _lane_gather(ev[j2 // 8], lane_consts[(2 * j2 + 1) % L])
                j2s = jnp.full((L,), j2, jnp.int32)
                for g in range(4):
                    w = plsc.load_gather(ttb_v, [j2s, tvecs[g]])
                    acc_e[g] = acc_e[g] + _unpack_lo(w) * e_even
                    acc_o[g] = acc_o[g] + _unpack_hi(w) * e_odd
            for g in range(4):
                itv = plsc.load_gather(inv_t_v, [tvecs[g]])
                d = (acc_e[g] + acc_o[g]) * itv * inv_e_b
                s = jnp.maximum(2.0 - 2.0 * d, 0.0)
                out_v[pl.ds(off + g * L, L)] = s * _rsqrt(jnp.maximum(s, 1e-30))
            return 0
        lax.fori_loop(0, C, b_body, 0)

        pltpu.sync_copy(out_v.at[pl.ds(0, C * NEG)],
                        out_hbm.at[pl.ds(base * NEG, C * NEG)])
        return 0
    lax.fori_loop(0, NCHUNK, chunk_body, 0)


def kernel(ent, ent_type, batch_type, ent_table, type_table):
    tt = jnp.pad(type_table.astype(jnp.float32).T, ((0, 0), (0, NTP - NT)))
    ttb = lax.bitcast_convert_type(
        tt.astype(jnp.bfloat16).reshape(NPAIR, 2, NTP).transpose(0, 2, 1),
        jnp.int32)
    ent_i = ent.astype(jnp.int32)
    et_flat = ent_type.astype(jnp.int32).reshape(-1)

    mesh = plsc.VectorSubcoreMesh(core_axis_name="c", subcore_axis_name="s",
                                  num_cores=NC, num_subcores=NS)
    run = pl.kernel(
        _sc_kernel,
        out_type=jax.ShapeDtypeStruct((B * NEG,), jnp.float32),
        mesh=mesh,
        compiler_params=pltpu.CompilerParams(needs_layout_passes=False,
                                             use_tc_tiling_on_sc=False),
        scratch_types=[
            pltpu.VMEM((NPAIR, NTP), jnp.int32),
            pltpu.VMEM((NTP,), jnp.float32),
            pltpu.VMEM((C,), jnp.int32),
            pltpu.VMEM((C, DIM), jnp.float32),
            pltpu.VMEM((C * NEG + L,), jnp.int32),
            pltpu.VMEM((C * NEG + L,), jnp.float32),
            pltpu.SemaphoreType.DMA,
        ],
    )
    out = run(ttb, ent_i, et_flat, ent_table.astype(jnp.float32))
    return out.reshape(B, NEG)


# speedup vs baseline: 5.6540x; 1.0012x over previous
import jax
import jax.numpy as jnp
from jax import lax
from jax.experimental import pallas as pl
from jax.experimental.pallas import tpu as pltpu
from jax.experimental.pallas import tpu_sc as plsc

NC, NS, L = 2, 16, 16
NW = NC * NS
B = 16384
NEG = 50
DIM = 64
NT = 1000
NTP = 1024
NPAIR = DIM // 2
BPW = B // NW
C = 128
NCHUNK = BPW // C

def _rsqrt(x):
    i = lax.bitcast_convert_type(x, jnp.int32)
    i = jnp.int32(0x5F3759DF) - lax.shift_right_logical(i, 1)
    y = lax.bitcast_convert_type(i, jnp.float32)
    for _ in range(3):
        y = y * (1.5 - 0.5 * x * y * y)
    return y


_GATHER_DNUMS = lax.GatherDimensionNumbers(
    offset_dims=(), collapsed_slice_dims=(0,), start_index_map=(0,))


def _lane_gather(vec, idx):
    return lax.gather(vec, idx[:, None], _GATHER_DNUMS, (1,),
                      mode=lax.GatherScatterMode.PROMISE_IN_BOUNDS)


def _unpack_lo(w):
    return lax.bitcast_convert_type(lax.shift_left(w, 16), jnp.float32)


def _unpack_hi(w):
    return lax.bitcast_convert_type(
        jnp.bitwise_and(w, jnp.int32(-65536)), jnp.float32)


def _sc_kernel(ttb_hbm, ent_hbm, et_hbm, table_hbm, out_hbm,
               ttb_v, inv_t_v, idx_v, rows_v, et_v, out_v, sem):
    wid = lax.axis_index("s") * NC + lax.axis_index("c")

    with jax.named_scope("stage_tt"):
        pltpu.sync_copy(ttb_hbm, ttb_v)

    def tnorm_body(cb, _):
        acc = jnp.zeros((L,), jnp.float32)
        for j2 in range(NPAIR):
            w = ttb_v[j2, pl.ds(cb * L, L)]
            lo = _unpack_lo(w)
            hi = _unpack_hi(w)
            acc = acc + lo * lo + hi * hi
        inv_t_v[pl.ds(cb * L, L)] = _rsqrt(jnp.maximum(acc, 1e-24))
        return 0
    with jax.named_scope("tnorm"):
        lax.fori_loop(0, NTP // L, tnorm_body, 0)

    lane_consts = [jnp.full((L,), i, jnp.int32) for i in range(L)]

    def chunk_body(cnk, _):
        base = wid * BPW + cnk * C
        pltpu.sync_copy(ent_hbm.at[pl.ds(base, C)], idx_v)
        cp = pltpu.async_copy(table_hbm.at[idx_v], rows_v, sem)
        pltpu.sync_copy(et_hbm.at[pl.ds(base * NEG, C * NEG)],
                        et_v.at[pl.ds(0, C * NEG)])
        et_v[pl.ds(C * NEG, L)] = jnp.zeros((L,), jnp.int32)
        cp.wait()

        def b_body(b, _):
            ev = [rows_v[b, pl.ds(k * L, L)] for k in range(DIM // L)]
            sq = ev[0] * ev[0] + ev[1] * ev[1] + ev[2] * ev[2] + ev[3] * ev[3]
            tot = _lane_gather(plsc.cumsum(sq), lane_consts[L - 1])
            inv_e_b = _rsqrt(jnp.maximum(tot, 1e-24))

            off = b * NEG
            tvecs = [et_v[pl.ds(off + g * L, L)] for g in range(4)]
            acc_e = [jnp.zeros((L,), jnp.float32) for _ in range(4)]
            acc_o = [jnp.zeros((L,), jnp.float32) for _ in range(4)]
            for j2 in range(NPAIR):
                e_even = _lane_gather(ev[j2 // 8], lane_consts[(2 * j2) % L])
                e_odd = _lane_gather(ev[j2 // 8], lane_consts[(2 * j2 + 1) % L])
                j2s = jnp.full((L,), j2, jnp.int32)
                for g in range(4):
                    w = plsc.load_gather(ttb_v, [j2s, tvecs[g]])
                    acc_e[g] = acc_e[g] + _unpack_lo(w) * e_even
                    acc_o[g] = acc_o[g] + _unpack_hi(w) * e_odd
            for g in range(4):
                itv = plsc.load_gather(inv_t_v, [tvecs[g]])
                d = (acc_e[g] + acc_o[g]) * itv * inv_e_b
                s = jnp.maximum(2.0 - 2.0 * d, 0.0)
                out_v[pl.ds(off + g * L, L)] = s * _rsqrt(jnp.maximum(s, 1e-30))
            return 0
        with jax.named_scope("bloop"):
            lax.fori_loop(0, C, b_body, 0)

        pltpu.sync_copy(out_v.at[pl.ds(0, C * NEG)],
                        out_hbm.at[pl.ds(base * NEG, C * NEG)])
        return 0
    lax.fori_loop(0, NCHUNK, chunk_body, 0)


def kernel(ent, ent_type, batch_type, ent_table, type_table):
    tt = jnp.pad(type_table.astype(jnp.float32).T, ((0, 0), (0, NTP - NT)))
    ttb = lax.bitcast_convert_type(
        tt.astype(jnp.bfloat16).reshape(NPAIR, 2, NTP).transpose(0, 2, 1),
        jnp.int32)
    ent_i = ent.astype(jnp.int32)
    et_flat = ent_type.astype(jnp.int32).reshape(-1)

    mesh = plsc.VectorSubcoreMesh(core_axis_name="c", subcore_axis_name="s",
                                  num_cores=NC, num_subcores=NS)
    run = pl.kernel(
        _sc_kernel,
        out_type=jax.ShapeDtypeStruct((B * NEG,), jnp.float32),
        mesh=mesh,
        compiler_params=pltpu.CompilerParams(needs_layout_passes=False,
                                             use_tc_tiling_on_sc=False),
        scratch_types=[
            pltpu.VMEM((NPAIR, NTP), jnp.int32),
            pltpu.VMEM((NTP,), jnp.float32),
            pltpu.VMEM((C,), jnp.int32),
            pltpu.VMEM((C, DIM), jnp.float32),
            pltpu.VMEM((C * NEG + L,), jnp.int32),
            pltpu.VMEM((C * NEG + L,), jnp.float32),
            pltpu.SemaphoreType.DMA,
        ],
    )
    out = run(ttb, ent_i, et_flat, ent_table.astype(jnp.float32))
    return out.reshape(B, NEG)
